# Initial kernel scaffold; baseline (speedup 1.0000x reference)
import jax
import jax.numpy as jnp
from jax.experimental import pallas as pl


def kernel(x, edge_index, W1, b1, W2, b2):
    raise NotImplementedError("write your pallas kernel here")


# speedup vs baseline: 15.3350x; 15.3350x over previous
import dataclasses
import functools

import jax
import jax.numpy as jnp
from jax import lax
from jax.experimental import pallas as pl
from jax.experimental.pallas import tpu as pltpu
from jax.experimental.pallas import tpu_sc as plsc

N = 10000
E = 320000
D = 128
NC, NS, L = 2, 16, 16
NW = NC * NS
CHUNK = 128
CPT = 80
EPT = CHUNK * CPT
EPAD = NW * EPT
NDEG = 10240
ACC_ROWS = NDEG
ZROWS = ACC_ROWS // NS
DRAIN = ACC_ROWS // NS
RB = 1000
GRID = N // RB

_MESH = dict(core_axis_name="c", subcore_axis_name="s")

_SC_PARAMS = pltpu.CompilerParams()
if "needs_layout_passes" in pltpu.CompilerParams.__dataclass_fields__:
    _SC_PARAMS = dataclasses.replace(_SC_PARAMS, needs_layout_passes=False)


def _sc_degree(dst_flat):
    mesh = plsc.VectorSubcoreMesh(**_MESH)

    @functools.partial(
        pl.kernel, mesh=mesh, compiler_params=_SC_PARAMS,
        out_type=jax.ShapeDtypeStruct((NW, NDEG), jnp.float32),
        scratch_types=[pltpu.VMEM((EPT,), jnp.int32),
                       pltpu.VMEM((NDEG,), jnp.float32)])
    def k(dst_hbm, out_hbm, idx_v, hist_v):
        w = lax.axis_index("c") * NS + lax.axis_index("s")
        pltpu.sync_copy(dst_hbm.at[w], idx_v)
        zeros16 = jnp.zeros((L,), jnp.float32)

        @pl.loop(0, NDEG // L)
        def _(i):
            hist_v[pl.ds(i * L, L)] = zeros16

        ones16 = jnp.ones((L,), jnp.float32)

        @pl.loop(0, EPT // L)
        def _(i):
            idx = idx_v[pl.ds(i * L, L)]
            plsc.addupdate_scatter(hist_v, [idx], ones16)

        pltpu.sync_copy(hist_v, out_hbm.at[w])

    return k(dst_flat)


def _sc_propagate(g, src3, dst3, zrows):
    mesh = plsc.VectorSubcoreMesh(**_MESH)

    @functools.partial(
        pl.kernel, mesh=mesh,
        out_type=jax.ShapeDtypeStruct((NC, ACC_ROWS, D), jnp.float32),
        scratch_types=[
            pltpu.VMEM((CPT, CHUNK), jnp.int32),
            pltpu.VMEM((CPT, CHUNK), jnp.int32),
            pltpu.VMEM((CHUNK, D), jnp.float32),
            pltpu.VMEM_SHARED((ACC_ROWS, D), jnp.float32),
        ])
    def k(g_hbm, src_hbm, dst_hbm, z_hbm, out_hbm, src_v, dst_v, rows_v, acc):
        c = lax.axis_index("c")
        s = lax.axis_index("s")
        w = c * NS + s
        pltpu.sync_copy(src_hbm.at[w], src_v)
        pltpu.sync_copy(dst_hbm.at[w], dst_v)
        pltpu.sync_copy(z_hbm, acc.at[pl.ds(s * ZROWS, ZROWS)])
        plsc.subcore_barrier()

        @pl.loop(0, CPT)
        def _(i):
            pltpu.sync_copy(g_hbm.at[src_v.at[i]], rows_v)
            pltpu.sync_copy(rows_v, acc.at[dst_v.at[i]], add=True)

        plsc.subcore_barrier()
        pltpu.sync_copy(acc.at[pl.ds(s * DRAIN, DRAIN)],
                        out_hbm.at[c, pl.ds(s * DRAIN, DRAIN)])

    return k(g, src3, dst3, zrows)


def _dinv(d_ref):
    deg = jnp.sum(d_ref[...], axis=0) + 1.0
    return lax.rsqrt(deg)


def _tc_first(x, W1, degp):
    def body(x_ref, w_ref, d_ref, g_ref):
        dinv = _dinv(d_ref)
        h = jnp.dot(x_ref[...], w_ref[...],
                    preferred_element_type=jnp.float32,
                    precision=lax.Precision.HIGHEST)
        g_ref[...] = dinv * h

    return pl.pallas_call(
        body,
        grid=(GRID,),
        in_specs=[pl.BlockSpec((RB, D), lambda i: (i, 0)),
                  pl.BlockSpec((D, D), lambda i: (0, 0)),
                  pl.BlockSpec((NW, RB, 1), lambda i: (0, i, 0))],
        out_specs=pl.BlockSpec((RB, D), lambda i: (i, 0)),
        out_shape=jax.ShapeDtypeStruct((N, D), jnp.float32),
    )(x, W1, degp)


def _tc_mid(P, g, degp, b, W):
    def body(p_ref, g_ref, d_ref, b_ref, w_ref, o_ref):
        dinv = _dinv(d_ref)
        ssum = p_ref[0] + p_ref[1] + g_ref[...]
        h = jax.nn.relu(dinv * ssum + b_ref[...])
        o_ref[...] = dinv * jnp.dot(h, w_ref[...],
                                    preferred_element_type=jnp.float32,
                                    precision=lax.Precision.HIGHEST)

    return pl.pallas_call(
        body,
        grid=(GRID,),
        in_specs=[pl.BlockSpec((NC, RB, D), lambda i: (0, i, 0)),
                  pl.BlockSpec((RB, D), lambda i: (i, 0)),
                  pl.BlockSpec((NW, RB, 1), lambda i: (0, i, 0)),
                  pl.BlockSpec((1, D), lambda i: (0, 0)),
                  pl.BlockSpec((D, D), lambda i: (0, 0))],
        out_specs=pl.BlockSpec((RB, D), lambda i: (i, 0)),
        out_shape=jax.ShapeDtypeStruct((N, D), jnp.float32),
    )(P, g, degp, b, W)


def _tc_last(P, g, degp, b):
    def body(p_ref, g_ref, d_ref, b_ref, o_ref):
        dinv = _dinv(d_ref)
        ssum = p_ref[0] + p_ref[1] + g_ref[...]
        o_ref[...] = jax.nn.relu(dinv * ssum + b_ref[...])

    return pl.pallas_call(
        body,
        grid=(GRID,),
        in_specs=[pl.BlockSpec((NC, RB, D), lambda i: (0, i, 0)),
                  pl.BlockSpec((RB, D), lambda i: (i, 0)),
                  pl.BlockSpec((NW, RB, 1), lambda i: (0, i, 0)),
                  pl.BlockSpec((1, D), lambda i: (0, 0))],
        out_specs=pl.BlockSpec((RB, D), lambda i: (i, 0)),
        out_shape=jax.ShapeDtypeStruct((N, D), jnp.float32),
    )(P, g, degp, b)


def kernel(x, edge_index, W1, b1, W2, b2):
    ei = edge_index.astype(jnp.int32)
    src, dst = ei[0], ei[1]
    npad = EPAD - E
    pad = jnp.arange(npad, dtype=jnp.int32)
    src3 = jnp.concatenate([src, (pad * 67) % N]).reshape(NW, CPT, CHUNK)
    dst3 = jnp.concatenate([dst, N + (pad % (ACC_ROWS - N))]).reshape(
        NW, CPT, CHUNK)
    zrows = jnp.zeros((ZROWS, D), jnp.float32)

    degp = _sc_degree(dst3.reshape(NW, EPT))
    degp3 = degp.reshape(NW, NDEG, 1)
    b1r = b1.reshape(1, D)
    b2r = b2.reshape(1, D)

    g1 = _tc_first(x, W1, degp3)
    P1 = _sc_propagate(g1, src3, dst3, zrows)
    g2 = _tc_mid(P1, g1, degp3, b1r, W2)
    P2 = _sc_propagate(g2, src3, dst3, zrows)
    return _tc_last(P2, g2, degp3, b2r)


# speedup vs baseline: 16.9052x; 1.1024x over previous
import dataclasses
import functools

import jax
import jax.numpy as jnp
from jax import lax
from jax.experimental import pallas as pl
from jax.experimental.pallas import tpu as pltpu
from jax.experimental.pallas import tpu_sc as plsc

N = 10000
E = 320000
D = 128
NC, NS, L = 2, 16, 16
NW = NC * NS
CHUNK = 128
CPT = 80
PH = 2
CPP = CPT // PH
EPT = CHUNK * CPT
EPAD = NW * EPT
NDEG = 10240
ACC_ROWS = NDEG
ZROWS = ACC_ROWS // NS
DRAIN = ACC_ROWS // NS
RB = 1000
GRID = N // RB

_MESH = dict(core_axis_name="c", subcore_axis_name="s")

_SC_PARAMS = pltpu.CompilerParams()
if "needs_layout_passes" in pltpu.CompilerParams.__dataclass_fields__:
    _SC_PARAMS = dataclasses.replace(_SC_PARAMS, needs_layout_passes=False)


def _sc_degree(dst_flat):
    mesh = plsc.VectorSubcoreMesh(**_MESH)

    @functools.partial(
        pl.kernel, mesh=mesh, compiler_params=_SC_PARAMS,
        out_type=jax.ShapeDtypeStruct((NW, NDEG), jnp.float32),
        scratch_types=[pltpu.VMEM((EPT,), jnp.int32),
                       pltpu.VMEM((NDEG,), jnp.float32)])
    def k(dst_hbm, out_hbm, idx_v, hist_v):
        w = lax.axis_index("c") * NS + lax.axis_index("s")
        pltpu.sync_copy(dst_hbm.at[w], idx_v)
        zeros16 = jnp.zeros((L,), jnp.float32)

        @pl.loop(0, NDEG // L)
        def _(i):
            hist_v[pl.ds(i * L, L)] = zeros16

        ones16 = jnp.ones((L,), jnp.float32)

        @pl.loop(0, EPT // L)
        def _(i):
            idx = idx_v[pl.ds(i * L, L)]
            plsc.addupdate_scatter(hist_v, [idx], ones16)

        pltpu.sync_copy(hist_v, out_hbm.at[w])

    return k(dst_flat)


def _sc_propagate(g, src3, dst3, zrows):
    mesh = plsc.VectorSubcoreMesh(**_MESH)

    @functools.partial(
        pl.kernel, mesh=mesh,
        out_type=jax.ShapeDtypeStruct((NC, ACC_ROWS, D), jnp.float32),
        scratch_types=[
            pltpu.VMEM((CPP, CHUNK), jnp.int32),
            pltpu.VMEM((CPP, CHUNK), jnp.int32),
            pltpu.VMEM((CHUNK, D), jnp.float32),
            pltpu.VMEM((CHUNK, D), jnp.float32),
            pltpu.VMEM_SHARED((ACC_ROWS, D), jnp.float32),
            pltpu.SemaphoreType.DMA,
            pltpu.SemaphoreType.DMA,
            pltpu.SemaphoreType.DMA,
            pltpu.SemaphoreType.DMA,
        ])
    def k(g_hbm, src_hbm, dst_hbm, z_hbm, out_hbm,
          src_v, dst_v, rows_a, rows_b, acc, gs_a, gs_b, ss_a, ss_b):
        c = lax.axis_index("c")
        s = lax.axis_index("s")
        w = c * NS + s
        pltpu.sync_copy(z_hbm, acc.at[pl.ds(s * ZROWS, ZROWS)])
        plsc.subcore_barrier()

        def g_start(i, buf, sem):
            pltpu.async_copy(g_hbm.at[src_v.at[i]], buf, sem)

        def g_wait(i, buf, sem):
            pltpu.make_async_copy(g_hbm.at[src_v.at[i]], buf, sem).wait()

        def s_start(i, buf, sem):
            pltpu.async_copy(buf, acc.at[dst_v.at[i]], sem, add=True)

        def s_wait(i, buf, sem):
            pltpu.make_async_copy(buf, acc.at[dst_v.at[i]], sem).wait()

        for p in range(PH):
            pltpu.sync_copy(src_hbm.at[w, pl.ds(p * CPP, CPP)], src_v)
            pltpu.sync_copy(dst_hbm.at[w, pl.ds(p * CPP, CPP)], dst_v)
            g_start(0, rows_a, gs_a)
            g_start(1, rows_b, gs_b)

            @pl.loop(0, CPP - 2, step=2)
            def _(i):
                g_wait(i, rows_a, gs_a)
                s_start(i, rows_a, ss_a)
                g_wait(i + 1, rows_b, gs_b)
                s_start(i + 1, rows_b, ss_b)
                s_wait(i, rows_a, ss_a)
                g_start(i + 2, rows_a, gs_a)
                s_wait(i + 1, rows_b, ss_b)
                g_start(i + 3, rows_b, gs_b)

            g_wait(CPP - 2, rows_a, gs_a)
            s_start(CPP - 2, rows_a, ss_a)
            g_wait(CPP - 1, rows_b, gs_b)
            s_start(CPP - 1, rows_b, ss_b)
            s_wait(CPP - 2, rows_a, ss_a)
            s_wait(CPP - 1, rows_b, ss_b)

        plsc.subcore_barrier()
        pltpu.sync_copy(acc.at[pl.ds(s * DRAIN, DRAIN)],
                        out_hbm.at[c, pl.ds(s * DRAIN, DRAIN)])

    return k(g, src3, dst3, zrows)


def _dinv(d_ref):
    deg = jnp.sum(d_ref[...], axis=0) + 1.0
    return lax.rsqrt(deg)


def _tc_first(x, W1, degp):
    def body(x_ref, w_ref, d_ref, g_ref):
        dinv = _dinv(d_ref)
        h = jnp.dot(x_ref[...], w_ref[...],
                    preferred_element_type=jnp.float32,
                    precision=lax.Precision.HIGHEST)
        g_ref[...] = dinv * h

    return pl.pallas_call(
        body,
        grid=(GRID,),
        in_specs=[pl.BlockSpec((RB, D), lambda i: (i, 0)),
                  pl.BlockSpec((D, D), lambda i: (0, 0)),
                  pl.BlockSpec((NW, RB, 1), lambda i: (0, i, 0))],
        out_specs=pl.BlockSpec((RB, D), lambda i: (i, 0)),
        out_shape=jax.ShapeDtypeStruct((N, D), jnp.float32),
    )(x, W1, degp)


def _tc_mid(P, g, degp, b, W):
    def body(p_ref, g_ref, d_ref, b_ref, w_ref, o_ref):
        dinv = _dinv(d_ref)
        ssum = p_ref[0] + p_ref[1] + g_ref[...]
        h = jax.nn.relu(dinv * ssum + b_ref[...])
        o_ref[...] = dinv * jnp.dot(h, w_ref[...],
                                    preferred_element_type=jnp.float32,
                                    precision=lax.Precision.HIGHEST)

    return pl.pallas_call(
        body,
        grid=(GRID,),
        in_specs=[pl.BlockSpec((NC, RB, D), lambda i: (0, i, 0)),
                  pl.BlockSpec((RB, D), lambda i: (i, 0)),
                  pl.BlockSpec((NW, RB, 1), lambda i: (0, i, 0)),
                  pl.BlockSpec((1, D), lambda i: (0, 0)),
                  pl.BlockSpec((D, D), lambda i: (0, 0))],
        out_specs=pl.BlockSpec((RB, D), lambda i: (i, 0)),
        out_shape=jax.ShapeDtypeStruct((N, D), jnp.float32),
    )(P, g, degp, b, W)


def _tc_last(P, g, degp, b):
    def body(p_ref, g_ref, d_ref, b_ref, o_ref):
        dinv = _dinv(d_ref)
        ssum = p_ref[0] + p_ref[1] + g_ref[...]
        o_ref[...] = jax.nn.relu(dinv * ssum + b_ref[...])

    return pl.pallas_call(
        body,
        grid=(GRID,),
        in_specs=[pl.BlockSpec((NC, RB, D), lambda i: (0, i, 0)),
                  pl.BlockSpec((RB, D), lambda i: (i, 0)),
                  pl.BlockSpec((NW, RB, 1), lambda i: (0, i, 0)),
                  pl.BlockSpec((1, D), lambda i: (0, 0))],
        out_specs=pl.BlockSpec((RB, D), lambda i: (i, 0)),
        out_shape=jax.ShapeDtypeStruct((N, D), jnp.float32),
    )(P, g, degp, b)


def kernel(x, edge_index, W1, b1, W2, b2):
    ei = edge_index.astype(jnp.int32)
    src, dst = ei[0], ei[1]
    npad = EPAD - E
    pad = jnp.arange(npad, dtype=jnp.int32)
    src3 = jnp.concatenate([src, (pad * 67) % N]).reshape(NW, CPT, CHUNK)
    dst3 = jnp.concatenate([dst, N + (pad % (ACC_ROWS - N))]).reshape(
        NW, CPT, CHUNK)
    zrows = jnp.zeros((ZROWS, D), jnp.float32)

    degp = _sc_degree(dst3.reshape(NW, EPT))
    degp3 = degp.reshape(NW, NDEG, 1)
    b1r = b1.reshape(1, D)
    b2r = b2.reshape(1, D)

    g1 = _tc_first(x, W1, degp3)
    P1 = _sc_propagate(g1, src3, dst3, zrows)
    g2 = _tc_mid(P1, g1, degp3, b1r, W2)
    P2 = _sc_propagate(g2, src3, dst3, zrows)
    return _tc_last(P2, g2, degp3, b2r)


# speedup vs baseline: 25.0280x; 1.4805x over previous
import dataclasses
import functools

import jax
import jax.numpy as jnp
from jax import lax
from jax.experimental import pallas as pl
from jax.experimental.pallas import tpu as pltpu
from jax.experimental.pallas import tpu_sc as plsc

N = 10000
E = 320000
D = 128
NC, NS, L = 2, 16, 16
NW = NC * NS
EPT = E // NW
CHUNK = 125
CPT = EPT // CHUNK
PH = 2
CPP = CPT // PH
NDEG = 16384
HR = NDEG // 128
HRT = HR // NS
ACC_ROWS = 10240
ZROWS = ACC_ROWS // NS
DRAIN = ACC_ROWS // NS
RB = 1000
GRID = N // RB

_MESH = dict(core_axis_name="c", subcore_axis_name="s")

_SC_PARAMS = pltpu.CompilerParams()
if "needs_layout_passes" in pltpu.CompilerParams.__dataclass_fields__:
    _SC_PARAMS = dataclasses.replace(_SC_PARAMS, needs_layout_passes=False)


def _sc_degree(dst_flat):
    mesh = plsc.VectorSubcoreMesh(**_MESH)

    @functools.partial(
        pl.kernel, mesh=mesh, compiler_params=_SC_PARAMS,
        out_type=jax.ShapeDtypeStruct((NC, HR, 128), jnp.float32),
        scratch_types=[pltpu.VMEM((EPT,), jnp.int32),
                       pltpu.VMEM((HR, 128), jnp.float32),
                       pltpu.VMEM((1, HR), jnp.int32),
                       pltpu.VMEM_SHARED((HR, 128), jnp.float32)])
    def k(dst_hbm, out_hbm, idx_v, hist_v, rowidx_v, shist):
        c = lax.axis_index("c")
        s = lax.axis_index("s")
        w = c * NS + s
        pltpu.sync_copy(dst_hbm.at[w], idx_v)
        zeros16 = jnp.zeros((L,), jnp.float32)

        @pl.loop(0, HR)
        def _(r):
            for j in range(128 // L):
                hist_v[r, pl.ds(j * L, L)] = zeros16

        pltpu.sync_copy(hist_v.at[pl.ds(s * HRT, HRT)],
                        shist.at[pl.ds(s * HRT, HRT)])
        iota16 = lax.iota(jnp.int32, L)
        for j in range(HR // L):
            rowidx_v[0, pl.ds(j * L, L)] = iota16 + j * L
        plsc.subcore_barrier()

        ones16 = jnp.ones((L,), jnp.float32)

        @pl.loop(0, EPT // L)
        def _(i):
            idx = idx_v[pl.ds(i * L, L)]
            plsc.addupdate_scatter(hist_v, [idx // 128, idx % 128], ones16)

        pltpu.sync_copy(hist_v, shist.at[rowidx_v.at[0]], add=True)
        plsc.subcore_barrier()
        pltpu.sync_copy(shist.at[pl.ds(s * HRT, HRT)],
                        out_hbm.at[c, pl.ds(s * HRT, HRT)])

    return k(dst_flat)


def _sc_propagate(g, src3, dst3, zrows):
    mesh = plsc.VectorSubcoreMesh(**_MESH)

    @functools.partial(
        pl.kernel, mesh=mesh,
        out_type=jax.ShapeDtypeStruct((NC, ACC_ROWS, D), jnp.float32),
        scratch_types=[
            pltpu.VMEM((CPP, CHUNK), jnp.int32),
            pltpu.VMEM((CPP, CHUNK), jnp.int32),
            pltpu.VMEM((CHUNK, D), jnp.float32),
            pltpu.VMEM((CHUNK, D), jnp.float32),
            pltpu.VMEM_SHARED((ACC_ROWS, D), jnp.float32),
            pltpu.SemaphoreType.DMA,
            pltpu.SemaphoreType.DMA,
            pltpu.SemaphoreType.DMA,
            pltpu.SemaphoreType.DMA,
        ])
    def k(g_hbm, src_hbm, dst_hbm, z_hbm, out_hbm,
          src_v, dst_v, rows_a, rows_b, acc, gs_a, gs_b, ss_a, ss_b):
        c = lax.axis_index("c")
        s = lax.axis_index("s")
        w = c * NS + s
        pltpu.sync_copy(z_hbm, acc.at[pl.ds(s * ZROWS, ZROWS)])
        plsc.subcore_barrier()

        def g_start(i, buf, sem):
            pltpu.async_copy(g_hbm.at[src_v.at[i]], buf, sem)

        def g_wait(i, buf, sem):
            pltpu.make_async_copy(g_hbm.at[src_v.at[i]], buf, sem).wait()

        def s_start(i, buf, sem):
            pltpu.async_copy(buf, acc.at[dst_v.at[i]], sem, add=True)

        def s_wait(i, buf, sem):
            pltpu.make_async_copy(buf, acc.at[dst_v.at[i]], sem).wait()

        for p in range(PH):
            pltpu.sync_copy(src_hbm.at[w, pl.ds(p * CPP, CPP)], src_v)
            pltpu.sync_copy(dst_hbm.at[w, pl.ds(p * CPP, CPP)], dst_v)
            g_start(0, rows_a, gs_a)
            g_start(1, rows_b, gs_b)

            @pl.loop(0, CPP - 2, step=2)
            def _(i):
                g_wait(i, rows_a, gs_a)
                s_start(i, rows_a, ss_a)
                g_wait(i + 1, rows_b, gs_b)
                s_start(i + 1, rows_b, ss_b)
                s_wait(i, rows_a, ss_a)
                g_start(i + 2, rows_a, gs_a)
                s_wait(i + 1, rows_b, ss_b)
                g_start(i + 3, rows_b, gs_b)

            g_wait(CPP - 2, rows_a, gs_a)
            s_start(CPP - 2, rows_a, ss_a)
            g_wait(CPP - 1, rows_b, gs_b)
            s_start(CPP - 1, rows_b, ss_b)
            s_wait(CPP - 2, rows_a, ss_a)
            s_wait(CPP - 1, rows_b, ss_b)

        plsc.subcore_barrier()
        pltpu.sync_copy(acc.at[pl.ds(s * DRAIN, DRAIN)],
                        out_hbm.at[c, pl.ds(s * DRAIN, DRAIN)])

    return k(g, src3, dst3, zrows)


def _dinv(d_ref):
    deg = jnp.sum(d_ref[...], axis=0) + 1.0
    return lax.rsqrt(deg)


def _tc_first(x, W1, degp):
    def body(x_ref, w_ref, d_ref, g_ref):
        dinv = _dinv(d_ref)
        h = jnp.dot(x_ref[...], w_ref[...],
                    preferred_element_type=jnp.float32,
                    precision=lax.Precision.HIGHEST)
        g_ref[...] = dinv * h

    return pl.pallas_call(
        body,
        grid=(GRID,),
        in_specs=[pl.BlockSpec((RB, D), lambda i: (i, 0)),
                  pl.BlockSpec((D, D), lambda i: (0, 0)),
                  pl.BlockSpec((NC, RB, 1), lambda i: (0, i, 0))],
        out_specs=pl.BlockSpec((RB, D), lambda i: (i, 0)),
        out_shape=jax.ShapeDtypeStruct((N, D), jnp.float32),
    )(x, W1, degp)


def _tc_mid(P, g, degp, b, W):
    def body(p_ref, g_ref, d_ref, b_ref, w_ref, o_ref):
        dinv = _dinv(d_ref)
        ssum = p_ref[0] + p_ref[1] + g_ref[...]
        h = jax.nn.relu(dinv * ssum + b_ref[...])
        o_ref[...] = dinv * jnp.dot(h, w_ref[...],
                                    preferred_element_type=jnp.float32,
                                    precision=lax.Precision.HIGHEST)

    return pl.pallas_call(
        body,
        grid=(GRID,),
        in_specs=[pl.BlockSpec((NC, RB, D), lambda i: (0, i, 0)),
                  pl.BlockSpec((RB, D), lambda i: (i, 0)),
                  pl.BlockSpec((NC, RB, 1), lambda i: (0, i, 0)),
                  pl.BlockSpec((1, D), lambda i: (0, 0)),
                  pl.BlockSpec((D, D), lambda i: (0, 0))],
        out_specs=pl.BlockSpec((RB, D), lambda i: (i, 0)),
        out_shape=jax.ShapeDtypeStruct((N, D), jnp.float32),
    )(P, g, degp, b, W)


def _tc_last(P, g, degp, b):
    def body(p_ref, g_ref, d_ref, b_ref, o_ref):
        dinv = _dinv(d_ref)
        ssum = p_ref[0] + p_ref[1] + g_ref[...]
        o_ref[...] = jax.nn.relu(dinv * ssum + b_ref[...])

    return pl.pallas_call(
        body,
        grid=(GRID,),
        in_specs=[pl.BlockSpec((NC, RB, D), lambda i: (0, i, 0)),
                  pl.BlockSpec((RB, D), lambda i: (i, 0)),
                  pl.BlockSpec((NC, RB, 1), lambda i: (0, i, 0)),
                  pl.BlockSpec((1, D), lambda i: (0, 0))],
        out_specs=pl.BlockSpec((RB, D), lambda i: (i, 0)),
        out_shape=jax.ShapeDtypeStruct((N, D), jnp.float32),
    )(P, g, degp, b)


def kernel(x, edge_index, W1, b1, W2, b2):
    ei = edge_index.astype(jnp.int32)
    src3 = ei[0].reshape(NW, CPT, CHUNK)
    dst3 = ei[1].reshape(NW, CPT, CHUNK)
    zrows = jnp.zeros((ZROWS, D), jnp.float32)

    degp = _sc_degree(ei[1].reshape(NW, EPT))
    degp3 = degp.reshape(NC, NDEG, 1)
    b1r = b1.reshape(1, D)
    b2r = b2.reshape(1, D)

    g1 = _tc_first(x, W1, degp3)
    P1 = _sc_propagate(g1, src3, dst3, zrows)
    g2 = _tc_mid(P1, g1, degp3, b1r, W2)
    P2 = _sc_propagate(g2, src3, dst3, zrows)
    return _tc_last(P2, g2, degp3, b2r)
